# baseline (speedup 1.0000x reference)
import functools

import jax
import jax.numpy as jnp
from jax import lax
from jax.experimental import pallas as pl
from jax.experimental.pallas import tpu as pltpu
from jax.experimental.pallas import tpu_sc as plsc

N_EDGES = 160000
N_TRIP = 160000
H = 128

NC = 2
NS = 16
NW = NC * NS


_BA = 1600


def _silu(v):
    return v * (1.0 / (1.0 + jnp.exp(-v)))


def _pre_body(x_ref, rbf_ref, wkj_ref, bkj_ref, wrbf_ref, xkj_ref):
    xb = x_ref[...]
    rbf_p = jnp.dot(rbf_ref[...], wrbf_ref[...], preferred_element_type=jnp.float32)
    xkj_ref[...] = _silu(jnp.dot(xb, wkj_ref[...],
                                 preferred_element_type=jnp.float32) + bkj_ref[...]) * rbf_p


def _pre_call(x, rbf, wkjT, bkj, wrbfT):
    n = x.shape[0]
    grid = (n // _BA,)
    row_spec = pl.BlockSpec((_BA, H), lambda i: (i, 0))
    full = lambda shape: pl.BlockSpec(shape, lambda i: tuple(0 for _ in shape))
    return pl.pallas_call(
        _pre_body,
        grid=grid,
        in_specs=[
            row_spec,
            pl.BlockSpec((_BA, 6), lambda i: (i, 0)),
            full((H, H)), full((1, H)), full((6, H)),
        ],
        out_specs=row_spec,
        out_shape=jax.ShapeDtypeStruct((n, H), jnp.float32),
    )(x, rbf, wkjT, bkj, wrbfT)


_BC = 640


def _einsum_body(g_ref, sbf_ref, wsbf_ref, w2_ref, msg_ref):
    g = g_ref[...]
    sbfp = jnp.dot(sbf_ref[...], wsbf_ref[...],
                   preferred_element_type=jnp.float32)
    parts = [g * sbfp[:, j:j + 1] for j in range(8)]
    g2 = jnp.concatenate(parts, axis=1).astype(jnp.bfloat16)
    msg_ref[...] = jnp.dot(g2, w2_ref[...], preferred_element_type=jnp.float32)


def _einsum_call(gathered, sbf, wsbfT, w2):
    nsbf = sbf.shape[0] // _BC - 1
    grid = (_T_PAD // _BC,)
    row_spec = pl.BlockSpec((_BC, H), lambda i: (i, 0))
    full = lambda shape: pl.BlockSpec(shape, lambda i: tuple(0 for _ in shape))
    return pl.pallas_call(
        _einsum_body,
        grid=grid,
        in_specs=[
            row_spec,
            pl.BlockSpec((_BC, 42), lambda i: (jnp.minimum(i, nsbf), 0)),
            full((42, 8)), full((8 * H, H)),
        ],
        out_specs=row_spec,
        out_shape=jax.ShapeDtypeStruct((_T_PAD, H), jnp.float32),
    )(gathered, sbf, wsbfT, w2)


def _post_body(agg_ref, x_ref,
               wji_ref, bji_ref,
               bw1_ref, bb1_ref, bw2_ref, bb2_ref,
               lw_ref, lb_ref,
               aw1a_ref, ab1a_ref, aw2a_ref, ab2a_ref,
               aw1b_ref, ab1b_ref, aw2b_ref, ab2b_ref,
               out_ref):
    dot = lambda a, w: jnp.dot(a, w[...], preferred_element_type=jnp.float32)
    x_ji = _silu(dot(x_ref[...], wji_ref) + bji_ref[...])
    h = x_ji + agg_ref[...]
    h = h + _silu(dot(_silu(dot(h, bw1_ref) + bb1_ref[...]), bw2_ref) + bb2_ref[...])
    h = _silu(dot(h, lw_ref) + lb_ref[...]) + x_ref[...]
    h = h + _silu(dot(_silu(dot(h, aw1a_ref) + ab1a_ref[...]), aw2a_ref) + ab2a_ref[...])
    h = h + _silu(dot(_silu(dot(h, aw1b_ref) + ab1b_ref[...]), aw2b_ref) + ab2b_ref[...])
    out_ref[...] = h


def _post_call(agg, x, mats, vecs):
    n = x.shape[0]
    grid = (n // _BA,)
    row_spec = pl.BlockSpec((_BA, H), lambda i: (i, 0))
    fullm = pl.BlockSpec((H, H), lambda i: (0, 0))
    fullv = pl.BlockSpec((1, H), lambda i: (0, 0))
    wji, bw1, bw2, lw, aw1a, aw2a, aw1b, aw2b = mats
    bji, bb1, bb2, lb, ab1a, ab2a, ab1b, ab2b = vecs
    ops = [wji, bji, bw1, bb1, bw2, bb2, lw, lb,
           aw1a, ab1a, aw2a, ab2a, aw1b, ab1b, aw2b, ab2b]
    specs = [fullm, fullv] * 8
    return pl.pallas_call(
        _post_body,
        grid=grid,
        in_specs=[row_spec, row_spec] + specs,
        out_specs=row_spec,
        out_shape=jax.ShapeDtypeStruct((n, H), jnp.float32),
    )(agg, x, *ops)


_T_PAD = 163840
_GCHUNK = 128
_G_PER_W = _T_PAD // NW
_G_NBODY = _G_PER_W // (2 * _GCHUNK)


def _sc_gather(table, idx):
    mesh = plsc.VectorSubcoreMesh(core_axis_name="c", subcore_axis_name="s")

    @functools.partial(
        pl.kernel, mesh=mesh,
        out_type=jax.ShapeDtypeStruct((_T_PAD, H), jnp.float32),
        scratch_types=[
            pltpu.VMEM((2 * _GCHUNK,), jnp.int32),
            pltpu.VMEM((2, _GCHUNK, H), jnp.float32),
            pltpu.SemaphoreType.DMA,
            pltpu.SemaphoreType.DMA,
            pltpu.SemaphoreType.DMA,
            pltpu.SemaphoreType.DMA,
            pltpu.SemaphoreType.DMA,
            pltpu.SemaphoreType.DMA,
        ],
    )
    def gather_k(table_hbm, idx_hbm, out_hbm, idxg, rows,
                 semi0, semi1, semg0, semg1, semo0, semo1):
        wid = lax.axis_index("s") * NC + lax.axis_index("c")
        base = wid * _G_PER_W

        def idx_src(g):
            return idx_hbm.at[pl.ds(pl.multiple_of(base + g * _GCHUNK, 8),
                                    _GCHUNK)]

        def out_dst(g):
            return out_hbm.at[pl.ds(pl.multiple_of(base + g * _GCHUNK, 8),
                                    _GCHUNK)]

        pltpu.async_copy(idx_src(0), idxg.at[pl.ds(0, _GCHUNK)], semi0)
        pltpu.async_copy(idx_src(1), idxg.at[pl.ds(_GCHUNK, _GCHUNK)], semi1)

        def body(i, _):
            for b in range(2):
                g = i * 2 + b
                io = b * _GCHUNK
                semi = semi0 if b == 0 else semi1
                semg = semg0 if b == 0 else semg1
                semo = semo0 if b == 0 else semo1
                pltpu.make_async_copy(
                    idx_src(g), idxg.at[pl.ds(io, _GCHUNK)], semi).wait()
                @pl.when(i > 0)
                def _():
                    pltpu.make_async_copy(rows.at[b], out_dst(g), semo).wait()
                pltpu.async_copy(
                    table_hbm.at[idxg.at[pl.ds(io, _GCHUNK)]],
                    rows.at[b], semg)
                @pl.when(i < _G_NBODY - 1)
                def _():
                    pltpu.async_copy(
                        idx_src(g + 2), idxg.at[pl.ds(io, _GCHUNK)], semi)
            for b in range(2):
                g = i * 2 + b
                semg = semg0 if b == 0 else semg1
                semo = semo0 if b == 0 else semo1
                pltpu.make_async_copy(
                    table_hbm.at[idxg.at[pl.ds(b * _GCHUNK, _GCHUNK)]],
                    rows.at[b], semg).wait()
                pltpu.async_copy(rows.at[b], out_dst(g), semo)
            return 0

        lax.fori_loop(0, _G_NBODY, body, 0)
        pltpu.make_async_copy(rows.at[0], out_dst(0), semo0).wait()
        pltpu.make_async_copy(rows.at[1], out_dst(1), semo1).wait()

    return gather_k(table, idx)


_SCHUNK = 128
_R = 11904
_NUNIT = 7
_ACC_ROWS = _R + 256
_E_PAD = NC * _NUNIT * _R
_S_PER_W = _T_PAD // NS
_NBODY = _S_PER_W // (2 * _SCHUNK)
_ZPS = _ACC_ROWS // NS
_WPS = _R // NS


def _sc_scatter(msg, idx):
    mesh = plsc.VectorSubcoreMesh(core_axis_name="c", subcore_axis_name="s")
    zeros_blk = jnp.zeros((_SCHUNK, H), jnp.float32)
    idx_pad = jnp.concatenate(
        [idx, jnp.full((_T_PAD - N_TRIP,), 200000, jnp.int32)])

    @functools.partial(
        pl.kernel, mesh=mesh,
        out_type=jax.ShapeDtypeStruct((_E_PAD, H), jnp.float32),
        scratch_types=[
            pltpu.VMEM((2 * _SCHUNK,), jnp.int32),
            pltpu.VMEM((_SCHUNK,), jnp.int32),
            pltpu.VMEM((_SCHUNK,), jnp.int32),
            pltpu.VMEM((2, _SCHUNK, H), jnp.float32),
            pltpu.VMEM_SHARED((_ACC_ROWS, H), jnp.float32),
            pltpu.SemaphoreType.DMA,
            pltpu.SemaphoreType.DMA,
            pltpu.SemaphoreType.DMA,
            pltpu.SemaphoreType.DMA,
            pltpu.SemaphoreType.DMA,
            pltpu.SemaphoreType.DMA,
        ],
    )
    def scatter_k(msg_hbm, idx_hbm, zeros_hbm, out_hbm,
                  idxg, relg0, relg1, msgb, acc,
                  semi0, semi1, semg0, semg1, sema0, sema1):
        c = lax.axis_index("c")
        s = lax.axis_index("s")
        tbase = s * _S_PER_W
        lanes = lax.iota(jnp.int32, 16)
        dump = _R + 16 * s + lanes

        def idx_src(g):
            return idx_hbm.at[pl.ds(pl.multiple_of(tbase + g * _SCHUNK, 8),
                                    _SCHUNK)]

        def msg_src(g):
            return msg_hbm.at[pl.ds(pl.multiple_of(tbase + g * _SCHUNK, 8),
                                    _SCHUNK)]

        def unit_body(p, _):
            u = p * NC + c
            rowbase = u * _R

            pltpu.sync_copy(zeros_hbm, msgb.at[0])
            def zchunk(j, _):
                pltpu.sync_copy(
                    msgb.at[0],
                    acc.at[pl.ds(pl.multiple_of(s * _ZPS + j * _SCHUNK, 8),
                                 _SCHUNK)])
                return 0
            lax.fori_loop(0, _ZPS // _SCHUNK, zchunk, 0)
            pltpu.sync_copy(
                msgb.at[0].at[pl.ds(0, _ZPS % _SCHUNK)],
                acc.at[pl.ds(pl.multiple_of(
                    s * _ZPS + (_ZPS // _SCHUNK) * _SCHUNK, 8),
                    _ZPS % _SCHUNK)])
            plsc.subcore_barrier()

            pltpu.async_copy(idx_src(0), idxg.at[pl.ds(0, _SCHUNK)], semi0)
            pltpu.async_copy(idx_src(1), idxg.at[pl.ds(_SCHUNK, _SCHUNK)],
                             semi1)

            def body(i, _):
                for b in range(2):
                    g = i * 2 + b
                    io = b * _SCHUNK
                    relg = relg0 if b == 0 else relg1
                    semi = semi0 if b == 0 else semi1
                    semg = semg0 if b == 0 else semg1
                    sema = sema0 if b == 0 else sema1
                    pltpu.make_async_copy(
                        idx_src(g), idxg.at[pl.ds(io, _SCHUNK)], semi).wait()
                    @pl.when(i > 0)
                    def _():
                        pltpu.make_async_copy(
                            msgb.at[b], acc.at[relg], sema).wait()
                    for k in range(_SCHUNK // 16):
                        v = idxg[pl.ds(pl.multiple_of(io + k * 16, 16), 16)]
                        rel = v - rowbase
                        ok = (rel >= 0) & (rel < _R)
                        relg[pl.ds(k * 16, 16)] = jnp.where(ok, rel, dump)
                    pltpu.async_copy(msg_src(g), msgb.at[b], semg)
                    @pl.when(i < _NBODY - 1)
                    def _():
                        pltpu.async_copy(
                            idx_src(g + 2), idxg.at[pl.ds(io, _SCHUNK)], semi)
                for b in range(2):
                    relg = relg0 if b == 0 else relg1
                    semg = semg0 if b == 0 else semg1
                    sema = sema0 if b == 0 else sema1
                    pltpu.make_async_copy(
                        msg_src(i), msgb.at[b], semg).wait()
                    pltpu.async_copy(msgb.at[b], acc.at[relg], sema,
                                     add=True)
                return 0

            lax.fori_loop(0, _NBODY, body, 0)
            pltpu.make_async_copy(msgb.at[0], acc.at[relg0], sema0).wait()
            pltpu.make_async_copy(msgb.at[1], acc.at[relg1], sema1).wait()
            plsc.subcore_barrier()

            pltpu.sync_copy(
                acc.at[pl.ds(pl.multiple_of(s * _WPS, 8), _WPS)],
                out_hbm.at[pl.ds(rowbase + s * _WPS, _WPS)])
            plsc.subcore_barrier()
            return 0

        lax.fori_loop(0, _NUNIT, unit_body, 0)

    return scatter_k(msg, idx_pad, zeros_blk)


def kernel(x, rbf, sbf, idx_kj, idx_ji, lin_rbf_w, lin_sbf_w, lin_ji_w,
           lin_ji_b, lin_kj_w, lin_kj_b, W, before_w1, before_b1, before_w2,
           before_b2, lin_w, lin_b, after_w1, after_b1, after_w2, after_b2):
    f32 = jnp.float32
    idx_kj = idx_kj.astype(jnp.int32)
    idx_ji = idx_ji.astype(jnp.int32)

    wjiT = lin_ji_w.T.astype(f32)
    wkjT = lin_kj_w.T.astype(f32)
    wrbfT = lin_rbf_w.T.astype(f32)
    wsbfT = lin_sbf_w.T.astype(f32)
    w2 = W.transpose(1, 2, 0).reshape(8 * H, H).astype(jnp.bfloat16)

    bji = lin_ji_b.reshape(1, H)
    bkj = lin_kj_b.reshape(1, H)

    x_kj = _pre_call(x, rbf, wkjT, bkj, wrbfT)
    idx_kj_pad = jnp.concatenate(
        [idx_kj, jnp.zeros((_T_PAD - N_TRIP,), jnp.int32)])
    gathered = _sc_gather(x_kj, idx_kj_pad)
    msg = _einsum_call(gathered, sbf, wsbfT, w2)
    agg = _sc_scatter(msg, idx_ji)

    mats = (wjiT, before_w1[0].T, before_w2[0].T, lin_w.T,
            after_w1[0].T, after_w2[0].T, after_w1[1].T, after_w2[1].T)
    vecs = (bji, before_b1[0].reshape(1, H), before_b2[0].reshape(1, H),
            lin_b.reshape(1, H),
            after_b1[0].reshape(1, H), after_b2[0].reshape(1, H),
            after_b1[1].reshape(1, H), after_b2[1].reshape(1, H))
    return _post_call(agg, x, mats, vecs)


# speedup vs baseline: 1.0013x; 1.0013x over previous
import functools

import jax
import jax.numpy as jnp
from jax import lax
from jax.experimental import pallas as pl
from jax.experimental.pallas import tpu as pltpu
from jax.experimental.pallas import tpu_sc as plsc

N_EDGES = 160000
N_TRIP = 160000
H = 128

NC = 2
NS = 16
NW = NC * NS


_BA = 1600


def _silu(v):
    return v * (1.0 / (1.0 + jnp.exp(-v)))


def _pre_body(x_ref, rbf_ref, wkj_ref, bkj_ref, wrbf_ref, xkj_ref):
    xb = x_ref[...]
    rbf_p = jnp.dot(rbf_ref[...], wrbf_ref[...], preferred_element_type=jnp.float32)
    xkj_ref[...] = _silu(jnp.dot(xb, wkj_ref[...],
                                 preferred_element_type=jnp.float32) + bkj_ref[...]) * rbf_p


def _pre_call(x, rbf, wkjT, bkj, wrbfT):
    n = x.shape[0]
    grid = (n // _BA,)
    row_spec = pl.BlockSpec((_BA, H), lambda i: (i, 0))
    full = lambda shape: pl.BlockSpec(shape, lambda i: tuple(0 for _ in shape))
    return pl.pallas_call(
        _pre_body,
        grid=grid,
        in_specs=[
            row_spec,
            pl.BlockSpec((_BA, 6), lambda i: (i, 0)),
            full((H, H)), full((1, H)), full((6, H)),
        ],
        out_specs=row_spec,
        out_shape=jax.ShapeDtypeStruct((n, H), jnp.float32),
    )(x, rbf, wkjT, bkj, wrbfT)


_BC = 640


def _einsum_body(g_ref, sbf_ref, wsbf_ref, w2_ref, msg_ref):
    g = g_ref[...]
    sbfp = jnp.dot(sbf_ref[...], wsbf_ref[...],
                   preferred_element_type=jnp.float32)
    parts = [g * sbfp[:, j:j + 1] for j in range(8)]
    g2 = jnp.concatenate(parts, axis=1).astype(jnp.bfloat16)
    msg_ref[...] = jnp.dot(g2, w2_ref[...], preferred_element_type=jnp.float32)


def _einsum_call(gathered, sbf, wsbfT, w2):
    nsbf = sbf.shape[0] // _BC - 1
    grid = (_T_PAD // _BC,)
    row_spec = pl.BlockSpec((_BC, H), lambda i: (i, 0))
    full = lambda shape: pl.BlockSpec(shape, lambda i: tuple(0 for _ in shape))
    return pl.pallas_call(
        _einsum_body,
        grid=grid,
        in_specs=[
            row_spec,
            pl.BlockSpec((_BC, 42), lambda i: (jnp.minimum(i, nsbf), 0)),
            full((42, 8)), full((8 * H, H)),
        ],
        out_specs=row_spec,
        out_shape=jax.ShapeDtypeStruct((_T_PAD, H), jnp.float32),
    )(gathered, sbf, wsbfT, w2)


def _post_body(agg_ref, x_ref,
               wji_ref, bji_ref,
               bw1_ref, bb1_ref, bw2_ref, bb2_ref,
               lw_ref, lb_ref,
               aw1a_ref, ab1a_ref, aw2a_ref, ab2a_ref,
               aw1b_ref, ab1b_ref, aw2b_ref, ab2b_ref,
               out_ref):
    dot = lambda a, w: jnp.dot(a, w[...], preferred_element_type=jnp.float32)
    x_ji = _silu(dot(x_ref[...], wji_ref) + bji_ref[...])
    h = x_ji + agg_ref[...]
    h = h + _silu(dot(_silu(dot(h, bw1_ref) + bb1_ref[...]), bw2_ref) + bb2_ref[...])
    h = _silu(dot(h, lw_ref) + lb_ref[...]) + x_ref[...]
    h = h + _silu(dot(_silu(dot(h, aw1a_ref) + ab1a_ref[...]), aw2a_ref) + ab2a_ref[...])
    h = h + _silu(dot(_silu(dot(h, aw1b_ref) + ab1b_ref[...]), aw2b_ref) + ab2b_ref[...])
    out_ref[...] = h


def _post_call(agg, x, mats, vecs):
    n = x.shape[0]
    grid = (n // _BA,)
    row_spec = pl.BlockSpec((_BA, H), lambda i: (i, 0))
    fullm = pl.BlockSpec((H, H), lambda i: (0, 0))
    fullv = pl.BlockSpec((1, H), lambda i: (0, 0))
    wji, bw1, bw2, lw, aw1a, aw2a, aw1b, aw2b = mats
    bji, bb1, bb2, lb, ab1a, ab2a, ab1b, ab2b = vecs
    ops = [wji, bji, bw1, bb1, bw2, bb2, lw, lb,
           aw1a, ab1a, aw2a, ab2a, aw1b, ab1b, aw2b, ab2b]
    specs = [fullm, fullv] * 8
    return pl.pallas_call(
        _post_body,
        grid=grid,
        in_specs=[row_spec, row_spec] + specs,
        out_specs=row_spec,
        out_shape=jax.ShapeDtypeStruct((n, H), jnp.float32),
    )(agg, x, *ops)


_T_PAD = 163840
_GCHUNK = 128
_G_PER_W = _T_PAD // NW
_G_NBODY = _G_PER_W // (2 * _GCHUNK)


def _sc_gather(table, idx):
    mesh = plsc.VectorSubcoreMesh(core_axis_name="c", subcore_axis_name="s")

    @functools.partial(
        pl.kernel, mesh=mesh,
        out_type=jax.ShapeDtypeStruct((_T_PAD, H), jnp.float32),
        scratch_types=[
            pltpu.VMEM((2 * _GCHUNK,), jnp.int32),
            pltpu.VMEM((2, _GCHUNK, H), jnp.float32),
            pltpu.SemaphoreType.DMA,
            pltpu.SemaphoreType.DMA,
            pltpu.SemaphoreType.DMA,
            pltpu.SemaphoreType.DMA,
            pltpu.SemaphoreType.DMA,
            pltpu.SemaphoreType.DMA,
        ],
    )
    def gather_k(table_hbm, idx_hbm, out_hbm, idxg, rows,
                 semi0, semi1, semg0, semg1, semo0, semo1):
        wid = lax.axis_index("s") * NC + lax.axis_index("c")
        base = wid * _G_PER_W

        def idx_src(g):
            return idx_hbm.at[pl.ds(pl.multiple_of(base + g * _GCHUNK, 8),
                                    _GCHUNK)]

        def out_dst(g):
            return out_hbm.at[pl.ds(pl.multiple_of(base + g * _GCHUNK, 8),
                                    _GCHUNK)]

        pltpu.async_copy(idx_src(0), idxg.at[pl.ds(0, _GCHUNK)], semi0)
        pltpu.async_copy(idx_src(1), idxg.at[pl.ds(_GCHUNK, _GCHUNK)], semi1)

        def body(i, _):
            for b in range(2):
                g = i * 2 + b
                io = b * _GCHUNK
                semi = semi0 if b == 0 else semi1
                semg = semg0 if b == 0 else semg1
                semo = semo0 if b == 0 else semo1
                pltpu.make_async_copy(
                    idx_src(g), idxg.at[pl.ds(io, _GCHUNK)], semi).wait()
                @pl.when(i > 0)
                def _():
                    pltpu.make_async_copy(rows.at[b], out_dst(g), semo).wait()
                pltpu.async_copy(
                    table_hbm.at[idxg.at[pl.ds(io, _GCHUNK)]],
                    rows.at[b], semg)
            for b in range(2):
                g = i * 2 + b
                io = b * _GCHUNK
                semi = semi0 if b == 0 else semi1
                semg = semg0 if b == 0 else semg1
                semo = semo0 if b == 0 else semo1
                pltpu.make_async_copy(
                    table_hbm.at[idxg.at[pl.ds(io, _GCHUNK)]],
                    rows.at[b], semg).wait()
                @pl.when(i < _G_NBODY - 1)
                def _():
                    pltpu.async_copy(
                        idx_src(g + 2), idxg.at[pl.ds(io, _GCHUNK)], semi)
                pltpu.async_copy(rows.at[b], out_dst(g), semo)
            return 0

        lax.fori_loop(0, _G_NBODY, body, 0)
        pltpu.make_async_copy(rows.at[0], out_dst(0), semo0).wait()
        pltpu.make_async_copy(rows.at[1], out_dst(1), semo1).wait()

    return gather_k(table, idx)


_SCHUNK = 128
_R = 11904
_NUNIT = 7
_ACC_ROWS = _R + 256
_E_PAD = NC * _NUNIT * _R
_S_PER_W = _T_PAD // NS
_NBODY = _S_PER_W // (2 * _SCHUNK)
_ZPS = _ACC_ROWS // NS
_WPS = _R // NS


def _sc_scatter(msg, idx):
    mesh = plsc.VectorSubcoreMesh(core_axis_name="c", subcore_axis_name="s")
    zeros_blk = jnp.zeros((_SCHUNK, H), jnp.float32)
    idx_pad = jnp.concatenate(
        [idx, jnp.full((_T_PAD - N_TRIP,), 200000, jnp.int32)])

    @functools.partial(
        pl.kernel, mesh=mesh,
        out_type=jax.ShapeDtypeStruct((_E_PAD, H), jnp.float32),
        scratch_types=[
            pltpu.VMEM((2 * _SCHUNK,), jnp.int32),
            pltpu.VMEM((_SCHUNK,), jnp.int32),
            pltpu.VMEM((_SCHUNK,), jnp.int32),
            pltpu.VMEM((2, _SCHUNK, H), jnp.float32),
            pltpu.VMEM_SHARED((_ACC_ROWS, H), jnp.float32),
            pltpu.SemaphoreType.DMA,
            pltpu.SemaphoreType.DMA,
            pltpu.SemaphoreType.DMA,
            pltpu.SemaphoreType.DMA,
            pltpu.SemaphoreType.DMA,
            pltpu.SemaphoreType.DMA,
        ],
    )
    def scatter_k(msg_hbm, idx_hbm, zeros_hbm, out_hbm,
                  idxg, relg0, relg1, msgb, acc,
                  semi0, semi1, semg0, semg1, sema0, sema1):
        c = lax.axis_index("c")
        s = lax.axis_index("s")
        tbase = s * _S_PER_W
        lanes = lax.iota(jnp.int32, 16)
        dump = _R + 16 * s + lanes

        def idx_src(g):
            return idx_hbm.at[pl.ds(pl.multiple_of(tbase + g * _SCHUNK, 8),
                                    _SCHUNK)]

        def msg_src(g):
            return msg_hbm.at[pl.ds(pl.multiple_of(tbase + g * _SCHUNK, 8),
                                    _SCHUNK)]

        def unit_body(p, _):
            u = p * NC + c
            rowbase = u * _R

            pltpu.sync_copy(zeros_hbm, msgb.at[0])
            def zchunk(j, _):
                pltpu.sync_copy(
                    msgb.at[0],
                    acc.at[pl.ds(pl.multiple_of(s * _ZPS + j * _SCHUNK, 8),
                                 _SCHUNK)])
                return 0
            lax.fori_loop(0, _ZPS // _SCHUNK, zchunk, 0)
            pltpu.sync_copy(
                msgb.at[0].at[pl.ds(0, _ZPS % _SCHUNK)],
                acc.at[pl.ds(pl.multiple_of(
                    s * _ZPS + (_ZPS // _SCHUNK) * _SCHUNK, 8),
                    _ZPS % _SCHUNK)])
            plsc.subcore_barrier()

            pltpu.async_copy(idx_src(0), idxg.at[pl.ds(0, _SCHUNK)], semi0)
            pltpu.async_copy(idx_src(1), idxg.at[pl.ds(_SCHUNK, _SCHUNK)],
                             semi1)

            def body(i, _):
                for b in range(2):
                    g = i * 2 + b
                    io = b * _SCHUNK
                    relg = relg0 if b == 0 else relg1
                    semi = semi0 if b == 0 else semi1
                    semg = semg0 if b == 0 else semg1
                    sema = sema0 if b == 0 else sema1
                    pltpu.make_async_copy(
                        idx_src(g), idxg.at[pl.ds(io, _SCHUNK)], semi).wait()
                    @pl.when(i > 0)
                    def _():
                        pltpu.make_async_copy(
                            msgb.at[b], acc.at[relg], sema).wait()
                    for k in range(_SCHUNK // 16):
                        v = idxg[pl.ds(pl.multiple_of(io + k * 16, 16), 16)]
                        rel = v - rowbase
                        ok = (rel >= 0) & (rel < _R)
                        relg[pl.ds(k * 16, 16)] = jnp.where(ok, rel, dump)
                    pltpu.async_copy(msg_src(g), msgb.at[b], semg)
                    @pl.when(i < _NBODY - 1)
                    def _():
                        pltpu.async_copy(
                            idx_src(g + 2), idxg.at[pl.ds(io, _SCHUNK)], semi)
                for b in range(2):
                    relg = relg0 if b == 0 else relg1
                    semg = semg0 if b == 0 else semg1
                    sema = sema0 if b == 0 else sema1
                    pltpu.make_async_copy(
                        msg_src(i), msgb.at[b], semg).wait()
                    pltpu.async_copy(msgb.at[b], acc.at[relg], sema,
                                     add=True)
                return 0

            lax.fori_loop(0, _NBODY, body, 0)
            pltpu.make_async_copy(msgb.at[0], acc.at[relg0], sema0).wait()
            pltpu.make_async_copy(msgb.at[1], acc.at[relg1], sema1).wait()
            plsc.subcore_barrier()

            pltpu.sync_copy(
                acc.at[pl.ds(pl.multiple_of(s * _WPS, 8), _WPS)],
                out_hbm.at[pl.ds(rowbase + s * _WPS, _WPS)])
            plsc.subcore_barrier()
            return 0

        lax.fori_loop(0, _NUNIT, unit_body, 0)

    return scatter_k(msg, idx_pad, zeros_blk)


def kernel(x, rbf, sbf, idx_kj, idx_ji, lin_rbf_w, lin_sbf_w, lin_ji_w,
           lin_ji_b, lin_kj_w, lin_kj_b, W, before_w1, before_b1, before_w2,
           before_b2, lin_w, lin_b, after_w1, after_b1, after_w2, after_b2):
    f32 = jnp.float32
    idx_kj = idx_kj.astype(jnp.int32)
    idx_ji = idx_ji.astype(jnp.int32)

    wjiT = lin_ji_w.T.astype(f32)
    wkjT = lin_kj_w.T.astype(f32)
    wrbfT = lin_rbf_w.T.astype(f32)
    wsbfT = lin_sbf_w.T.astype(f32)
    w2 = W.transpose(1, 2, 0).reshape(8 * H, H).astype(jnp.bfloat16)

    bji = lin_ji_b.reshape(1, H)
    bkj = lin_kj_b.reshape(1, H)

    x_kj = _pre_call(x, rbf, wkjT, bkj, wrbfT)
    idx_kj_pad = jnp.concatenate(
        [idx_kj, jnp.zeros((_T_PAD - N_TRIP,), jnp.int32)])
    gathered = _sc_gather(x_kj, idx_kj_pad)
    msg = _einsum_call(gathered, sbf, wsbfT, w2)
    agg = _sc_scatter(msg, idx_ji)

    mats = (wjiT, before_w1[0].T, before_w2[0].T, lin_w.T,
            after_w1[0].T, after_w2[0].T, after_w1[1].T, after_w2[1].T)
    vecs = (bji, before_b1[0].reshape(1, H), before_b2[0].reshape(1, H),
            lin_b.reshape(1, H),
            after_b1[0].reshape(1, H), after_b2[0].reshape(1, H),
            after_b1[1].reshape(1, H), after_b2[1].reshape(1, H))
    return _post_call(agg, x, mats, vecs)


# speedup vs baseline: 1.1366x; 1.1351x over previous
import functools

import jax
import jax.numpy as jnp
from jax import lax
from jax.experimental import pallas as pl
from jax.experimental.pallas import tpu as pltpu
from jax.experimental.pallas import tpu_sc as plsc

N_EDGES = 160000
N_TRIP = 160000
H = 128

NC = 2
NS = 16
NW = NC * NS


_BA = 6400


def _silu(v):
    return v * (1.0 / (1.0 + jnp.exp(-v)))


def _pre_body(x_ref, rbf_ref, wkj_ref, bkj_ref, wrbf_ref, xkj_ref):
    xb = x_ref[...]
    rbf_p = jnp.dot(rbf_ref[...], wrbf_ref[...], preferred_element_type=jnp.float32)
    xkj_ref[...] = _silu(jnp.dot(xb, wkj_ref[...],
                                 preferred_element_type=jnp.float32) + bkj_ref[...]) * rbf_p


def _pre_call(x, rbf, wkjT, bkj, wrbfT):
    n = x.shape[0]
    grid = (n // _BA,)
    row_spec = pl.BlockSpec((_BA, H), lambda i: (i, 0))
    full = lambda shape: pl.BlockSpec(shape, lambda i: tuple(0 for _ in shape))
    return pl.pallas_call(
        _pre_body,
        grid=grid,
        in_specs=[
            row_spec,
            pl.BlockSpec((_BA, 6), lambda i: (i, 0)),
            full((H, H)), full((1, H)), full((6, H)),
        ],
        out_specs=row_spec,
        out_shape=jax.ShapeDtypeStruct((n, H), jnp.float32),
    )(x, rbf, wkjT, bkj, wrbfT)


_BC = 1280


def _einsum_body(g_ref, sbf_ref, wsbf_ref, w2_ref, msg_ref):
    g = g_ref[...]
    sbfp = jnp.dot(sbf_ref[...], wsbf_ref[...],
                   preferred_element_type=jnp.float32)
    parts = [g * sbfp[:, j:j + 1] for j in range(8)]
    g2 = jnp.concatenate(parts, axis=1).astype(jnp.bfloat16)
    msg_ref[...] = jnp.dot(g2, w2_ref[...], preferred_element_type=jnp.float32)


def _einsum_call(gathered, sbf, wsbfT, w2):
    nsbf = sbf.shape[0] // _BC - 1
    grid = (_T_PAD // _BC,)
    row_spec = pl.BlockSpec((_BC, H), lambda i: (i, 0))
    full = lambda shape: pl.BlockSpec(shape, lambda i: tuple(0 for _ in shape))
    return pl.pallas_call(
        _einsum_body,
        grid=grid,
        in_specs=[
            row_spec,
            pl.BlockSpec((_BC, 42), lambda i: (jnp.minimum(i, nsbf), 0)),
            full((42, 8)), full((8 * H, H)),
        ],
        out_specs=row_spec,
        out_shape=jax.ShapeDtypeStruct((_T_PAD, H), jnp.float32),
    )(gathered, sbf, wsbfT, w2)


def _post_body(agg_ref, x_ref,
               wji_ref, bji_ref,
               bw1_ref, bb1_ref, bw2_ref, bb2_ref,
               lw_ref, lb_ref,
               aw1a_ref, ab1a_ref, aw2a_ref, ab2a_ref,
               aw1b_ref, ab1b_ref, aw2b_ref, ab2b_ref,
               out_ref):
    dot = lambda a, w: jnp.dot(a, w[...], preferred_element_type=jnp.float32)
    x_ji = _silu(dot(x_ref[...], wji_ref) + bji_ref[...])
    h = x_ji + agg_ref[...]
    h = h + _silu(dot(_silu(dot(h, bw1_ref) + bb1_ref[...]), bw2_ref) + bb2_ref[...])
    h = _silu(dot(h, lw_ref) + lb_ref[...]) + x_ref[...]
    h = h + _silu(dot(_silu(dot(h, aw1a_ref) + ab1a_ref[...]), aw2a_ref) + ab2a_ref[...])
    h = h + _silu(dot(_silu(dot(h, aw1b_ref) + ab1b_ref[...]), aw2b_ref) + ab2b_ref[...])
    out_ref[...] = h


def _post_call(agg, x, mats, vecs):
    n = x.shape[0]
    grid = (n // _BA,)
    row_spec = pl.BlockSpec((_BA, H), lambda i: (i, 0))
    fullm = pl.BlockSpec((H, H), lambda i: (0, 0))
    fullv = pl.BlockSpec((1, H), lambda i: (0, 0))
    wji, bw1, bw2, lw, aw1a, aw2a, aw1b, aw2b = mats
    bji, bb1, bb2, lb, ab1a, ab2a, ab1b, ab2b = vecs
    ops = [wji, bji, bw1, bb1, bw2, bb2, lw, lb,
           aw1a, ab1a, aw2a, ab2a, aw1b, ab1b, aw2b, ab2b]
    specs = [fullm, fullv] * 8
    return pl.pallas_call(
        _post_body,
        grid=grid,
        in_specs=[row_spec, row_spec] + specs,
        out_specs=row_spec,
        out_shape=jax.ShapeDtypeStruct((n, H), jnp.float32),
    )(agg, x, *ops)


_T_PAD = 163840
_GCHUNK = 128
_G_PER_W = _T_PAD // NW
_G_NBODY = _G_PER_W // (2 * _GCHUNK)


def _sc_gather(table, idx):
    mesh = plsc.VectorSubcoreMesh(core_axis_name="c", subcore_axis_name="s")

    @functools.partial(
        pl.kernel, mesh=mesh,
        out_type=jax.ShapeDtypeStruct((_T_PAD, H), jnp.float32),
        scratch_types=[
            pltpu.VMEM((2 * _GCHUNK,), jnp.int32),
            pltpu.VMEM((2, _GCHUNK, H), jnp.float32),
            pltpu.SemaphoreType.DMA,
            pltpu.SemaphoreType.DMA,
            pltpu.SemaphoreType.DMA,
            pltpu.SemaphoreType.DMA,
            pltpu.SemaphoreType.DMA,
            pltpu.SemaphoreType.DMA,
        ],
    )
    def gather_k(table_hbm, idx_hbm, out_hbm, idxg, rows,
                 semi0, semi1, semg0, semg1, semo0, semo1):
        wid = lax.axis_index("s") * NC + lax.axis_index("c")
        base = wid * _G_PER_W

        def idx_src(g):
            return idx_hbm.at[pl.ds(pl.multiple_of(base + g * _GCHUNK, 8),
                                    _GCHUNK)]

        def out_dst(g):
            return out_hbm.at[pl.ds(pl.multiple_of(base + g * _GCHUNK, 8),
                                    _GCHUNK)]

        pltpu.async_copy(idx_src(0), idxg.at[pl.ds(0, _GCHUNK)], semi0)
        pltpu.async_copy(idx_src(1), idxg.at[pl.ds(_GCHUNK, _GCHUNK)], semi1)

        def body(i, _):
            for b in range(2):
                g = i * 2 + b
                io = b * _GCHUNK
                semi = semi0 if b == 0 else semi1
                semg = semg0 if b == 0 else semg1
                semo = semo0 if b == 0 else semo1
                pltpu.make_async_copy(
                    idx_src(g), idxg.at[pl.ds(io, _GCHUNK)], semi).wait()
                @pl.when(i > 0)
                def _():
                    pltpu.make_async_copy(rows.at[b], out_dst(g), semo).wait()
                pltpu.async_copy(
                    table_hbm.at[idxg.at[pl.ds(io, _GCHUNK)]],
                    rows.at[b], semg)
            for b in range(2):
                g = i * 2 + b
                io = b * _GCHUNK
                semi = semi0 if b == 0 else semi1
                semg = semg0 if b == 0 else semg1
                semo = semo0 if b == 0 else semo1
                pltpu.make_async_copy(
                    table_hbm.at[idxg.at[pl.ds(io, _GCHUNK)]],
                    rows.at[b], semg).wait()
                @pl.when(i < _G_NBODY - 1)
                def _():
                    pltpu.async_copy(
                        idx_src(g + 2), idxg.at[pl.ds(io, _GCHUNK)], semi)
                pltpu.async_copy(rows.at[b], out_dst(g), semo)
            return 0

        lax.fori_loop(0, _G_NBODY, body, 0)
        pltpu.make_async_copy(rows.at[0], out_dst(0), semo0).wait()
        pltpu.make_async_copy(rows.at[1], out_dst(1), semo1).wait()

    return gather_k(table, idx)


_SCHUNK = 128
_R = 12416
_NUNIT = 7
_ACC_ROWS = _R + 256
_E_PAD = NC * _NUNIT * _R
_S_PER_W = _T_PAD // NS
_IBLK = 2048
_SPARE = _S_PER_W + _SCHUNK
_ZPS = _ACC_ROWS // NS
_WPS = _R // NS


def _sc_scatter(msg, idx):
    mesh = plsc.VectorSubcoreMesh(core_axis_name="c", subcore_axis_name="s")
    zeros_blk = jnp.zeros((_ACC_ROWS, H), jnp.float32)
    idx_pad = jnp.concatenate(
        [idx, jnp.full((_T_PAD - N_TRIP,), 200000, jnp.int32)])

    @functools.partial(
        pl.kernel, mesh=mesh,
        out_type=jax.ShapeDtypeStruct((_E_PAD, H), jnp.float32),
        compiler_params=pltpu.CompilerParams(needs_layout_passes=False),
        scratch_types=[
            pltpu.VMEM((_IBLK,), jnp.int32),
            pltpu.VMEM((_SPARE + 16,), jnp.int32),
            pltpu.VMEM((_SCHUNK,), jnp.int32),
            pltpu.VMEM((_SCHUNK,), jnp.int32),
            pltpu.VMEM((_SCHUNK, H), jnp.float32),
            pltpu.VMEM_SHARED((_ACC_ROWS, H), jnp.float32),
            pltpu.SemaphoreType.DMA,
        ],
    )
    def scatter_k(msg_hbm, idx_hbm, zeros_hbm, out_hbm,
                  idxc, packbuf, pos_v, rel_v, msg_v, acc, sem):
        c = lax.axis_index("c")
        s = lax.axis_index("s")
        tbase = s * _S_PER_W
        lanes = lax.iota(jnp.int32, 16)
        dump = _R + 16 * s + lanes

        def unit_body(p, _):
            rowbase = (p * NC + c) * _R

            pltpu.sync_copy(
                zeros_hbm.at[pl.ds(pl.multiple_of(s * _ZPS, 8), _ZPS)],
                acc.at[pl.ds(pl.multiple_of(s * _ZPS, 8), _ZPS)])
            plsc.subcore_barrier()

            def blk_body(blk, cnt_vec):
                pltpu.sync_copy(
                    idx_hbm.at[pl.ds(pl.multiple_of(tbase + blk * _IBLK, 8),
                                     _IBLK)], idxc)
                def step(i, cv):
                    v = idxc[pl.ds(pl.multiple_of(i * 16, 16), 16)]
                    rel = v - rowbase
                    ok = (rel >= 0) & (rel < _R)
                    oki = ok.astype(jnp.int32)
                    slot = cv + jnp.cumsum(oki) - 1
                    slot = jnp.where(ok, slot, _SPARE + lanes)
                    packed = ((rel << 18)
                              | (tbase + blk * _IBLK + i * 16 + lanes))
                    plsc.store_scatter(packbuf, [slot], packed)
                    return cv + plsc.all_reduce_population_count(ok)
                return lax.fori_loop(0, _IBLK // 16, step, cnt_vec)

            cnt_vec = lax.fori_loop(0, _S_PER_W // _IBLK, blk_body,
                                    jnp.zeros((16,), jnp.int32))
            dpk = dump << 18
            for k in range(_SCHUNK // 16):
                plsc.store_scatter(packbuf, [cnt_vec + k * 16 + lanes], dpk)
            n = cnt_vec[0]

            def g_body(g, _):
                for k in range(_SCHUNK // 16):
                    pk = packbuf[pl.ds(
                        pl.multiple_of(g * _SCHUNK + k * 16, 16), 16)]
                    pos_v[pl.ds(k * 16, 16)] = pk & 0x3FFFF
                    rel_v[pl.ds(k * 16, 16)] = (pk >> 18) & 0x3FFF
                pltpu.async_copy(msg_hbm.at[pos_v], msg_v, sem).wait()
                pltpu.sync_copy(msg_v, acc.at[rel_v], add=True)
                return 0
            lax.fori_loop(0, (n + _SCHUNK - 1) >> 7, g_body, 0)
            plsc.subcore_barrier()

            pltpu.sync_copy(
                acc.at[pl.ds(pl.multiple_of(s * _WPS, 8), _WPS)],
                out_hbm.at[pl.ds(rowbase + s * _WPS, _WPS)])
            plsc.subcore_barrier()
            return 0

        lax.fori_loop(0, _NUNIT, unit_body, 0)

    return scatter_k(msg, idx_pad, zeros_blk)


def kernel(x, rbf, sbf, idx_kj, idx_ji, lin_rbf_w, lin_sbf_w, lin_ji_w,
           lin_ji_b, lin_kj_w, lin_kj_b, W, before_w1, before_b1, before_w2,
           before_b2, lin_w, lin_b, after_w1, after_b1, after_w2, after_b2):
    f32 = jnp.float32
    idx_kj = idx_kj.astype(jnp.int32)
    idx_ji = idx_ji.astype(jnp.int32)

    wjiT = lin_ji_w.T.astype(f32)
    wkjT = lin_kj_w.T.astype(f32)
    wrbfT = lin_rbf_w.T.astype(f32)
    wsbfT = lin_sbf_w.T.astype(f32)
    w2 = W.transpose(1, 2, 0).reshape(8 * H, H).astype(jnp.bfloat16)

    bji = lin_ji_b.reshape(1, H)
    bkj = lin_kj_b.reshape(1, H)

    x_kj = _pre_call(x, rbf, wkjT, bkj, wrbfT)
    idx_kj_pad = jnp.concatenate(
        [idx_kj, jnp.zeros((_T_PAD - N_TRIP,), jnp.int32)])
    gathered = _sc_gather(x_kj, idx_kj_pad)
    msg = _einsum_call(gathered, sbf, wsbfT, w2)
    agg = _sc_scatter(msg, idx_ji)

    mats = (wjiT, before_w1[0].T, before_w2[0].T, lin_w.T,
            after_w1[0].T, after_w2[0].T, after_w1[1].T, after_w2[1].T)
    vecs = (bji, before_b1[0].reshape(1, H), before_b2[0].reshape(1, H),
            lin_b.reshape(1, H),
            after_b1[0].reshape(1, H), after_b2[0].reshape(1, H),
            after_b1[1].reshape(1, H), after_b2[1].reshape(1, H))
    return _post_call(agg, x, mats, vecs)
